# baseline (speedup 1.0000x reference)
import jax
import jax.numpy as jnp
from jax.experimental import pallas as pl

N_NODES = 50000
N_EDGES = 800000
NUM_GRAPHS = 2048


def kernel(x, edge_index, edge_attr, batch, lin0_W, lin0_b, conv_Wf, conv_bf, conv_Ws, conv_bs, lin1_W, lin1_b, lin2_W, lin2_b, lin3_W, lin3_b):
    src = edge_index[0]
    dst = edge_index[1]
    h = jax.nn.relu(x @ lin0_W.T + lin0_b)
    for _ in range(2):
        x_i = jnp.take(h, dst, axis=0)
        x_j = jnp.take(h, src, axis=0)
        z = jnp.concatenate([x_i, x_j, edge_attr], axis=1)
        gate = jax.nn.sigmoid(z @ conv_Wf.T + conv_bf)
        core = jax.nn.softplus(z @ conv_Ws.T + conv_bs)
        msg = gate * core
        agg = jax.ops.segment_sum(msg, dst, num_segments=N_NODES)
        h = jax.nn.relu(h + agg)
    node_x = h @ lin1_W.T + lin1_b
    graph_x = jax.ops.segment_max(node_x, batch, num_segments=NUM_GRAPHS)
    graph_x = jax.nn.relu(graph_x)
    pred = jax.nn.relu(graph_x @ lin2_W.T + lin2_b)
    pred = pred @ lin3_W.T + lin3_b
    return pred
